# baseline (speedup 1.0000x reference)
import functools

import jax
import jax.numpy as jnp
from jax import lax
from jax.experimental import pallas as pl
from jax.experimental.pallas import tpu as pltpu
from jax.experimental.pallas import tpu_sc as plsc

NUM_ENTITIES = 1000000
EMBED_DIM = 64
BATCH = 16384
LANES = 128
NT = (NUM_ENTITIES + LANES - 1) // LANES
NBUF = 6
OUT_PAD = 16
SROWS = 48


@functools.cache
def _build_gather():
    info = plsc.get_sparse_core_info()
    nc, ns = info.num_cores, info.num_subcores
    nw = nc * ns
    tpw = (NT + nw - 1) // nw
    cap = BATCH + tpw * 15 + 16

    mesh = plsc.VectorSubcoreMesh(core_axis_name="c", subcore_axis_name="s")

    @functools.partial(
        pl.kernel,
        mesh=mesh,
        out_type=jax.ShapeDtypeStruct((BATCH + OUT_PAD, LANES), jnp.float32),
        scratch_types=[
            pltpu.VMEM((BATCH,), jnp.int32),
            pltpu.VMEM((BATCH,), jnp.int32),
            pltpu.VMEM((cap,), jnp.int32),
            pltpu.VMEM((cap,), jnp.int32),
            pltpu.VMEM((NBUF, EMBED_DIM, LANES), jnp.float32),
            pltpu.VMEM((SROWS, LANES), jnp.float32),
            pltpu.VMEM((SROWS,), jnp.int32),
            pltpu.SMEM((tpw,), jnp.int32),
            pltpu.SMEM((tpw,), jnp.int32),
            pltpu.SMEM((tpw,), jnp.int32),
            pltpu.SMEM((NBUF,), jnp.int32),
            pltpu.SemaphoreType.DMA((NBUF,)),
            pltpu.SemaphoreType.DMA,
        ],
        compiler_params=pltpu.CompilerParams(needs_layout_passes=False),
    )
    def gather(
        idx_hbm,
        table_t_hbm,
        out_hbm,
        idx_all,
        mpos,
        gid,
        gpos,
        bufs,
        stag,
        stagpos,
        hist,
        goff,
        gcur,
        pend,
        fsem,
        osem,
    ):
        wid = lax.axis_index("s") * nc + lax.axis_index("c")
        lo = wid * tpw
        hi = jnp.minimum(lo + tpw, NT)
        pltpu.sync_copy(idx_hbm, idx_all)

        iota = lax.iota(jnp.int32, 16)
        lane0 = iota < 1

        def p1(g, cnt):
            v = idx_all[pl.ds(pl.multiple_of(g * 16, 8), 16)]
            t = v >> 7
            m = (t >= lo) & (t < hi)
            s = plsc.cumsum(jnp.where(m, 1, 0))
            dst = cnt + s - 1
            plsc.store_scatter(idx_all, [dst], v, mask=m)
            plsc.store_scatter(mpos, [dst], iota + g * 16, mask=m)
            return cnt + s[15]

        cnt = lax.fori_loop(0, BATCH // 16, p1, jnp.int32(0))
        nchunks = (cnt + 15) >> 4

        def p2_clear(c, carry):
            hist[c] = 0
            return carry

        lax.fori_loop(0, tpw, p2_clear, None)

        def p2(g, carry):
            v = idx_all[pl.ds(pl.multiple_of(g * 16, 8), 16)]
            for b in range(16):

                @pl.when(g * 16 + b < cnt)
                def _():
                    t = (v[b] >> 7) - lo
                    hist[t] = hist[t] + 1

            return carry

        lax.fori_loop(0, nchunks, p2, None)

        def fire(c, s):
            t = pl.multiple_of((lo + c) << 7, LANES)
            pltpu.make_async_copy(
                table_t_hbm.at[:, pl.ds(t, LANES)], bufs.at[s], fsem.at[s]
            ).start()

        def p3(c, carry):
            run, nf = carry
            goff[c] = run
            gcur[c] = run

            @pl.when((hist[c] > 0) & (nf < NBUF))
            def _():
                fire(c, nf)
                pend[nf] = c

            nf2 = jnp.where((hist[c] > 0) & (nf < NBUF), nf + 1, nf)
            return run + ((hist[c] + 15) & ~15), nf2

        _, nfired = lax.fori_loop(0, tpw, p3, (jnp.int32(0), jnp.int32(0)))

        def p4(g, carry):
            v = idx_all[pl.ds(pl.multiple_of(g * 16, 8), 16)]
            p = mpos[pl.ds(pl.multiple_of(g * 16, 8), 16)]
            for b in range(16):

                @pl.when(g * 16 + b < cnt)
                def _():
                    t = (v[b] >> 7) - lo
                    slot = gcur[t]
                    gcur[t] = slot + 1
                    sv = jnp.full((16,), slot, jnp.int32)
                    plsc.store_scatter(
                        gid, [sv], jnp.full((16,), v[b], jnp.int32), mask=lane0
                    )
                    plsc.store_scatter(
                        gpos, [sv], jnp.full((16,), p[b], jnp.int32), mask=lane0
                    )

            return carry

        lax.fori_loop(0, nchunks, p4, None)

        def wait_one_flush():
            pltpu.make_async_copy(
                table_t_hbm.at[pl.ds(0, 16), pl.ds(0, LANES)],
                stag.at[pl.ds(0, 16), :],
                osem,
            ).wait()

        def flush(blk, started):
            @pl.when(started >= 1)
            def _():
                wait_one_flush()

            pltpu.make_async_copy(
                stag.at[pl.ds(blk, 16), :],
                out_hbm.at[stagpos.at[pl.ds(blk, 16)]],
                osem,
            ).start()
            return started + 1

        def extract(c, s, ocur, started):
            n = hist[c]
            o = goff[c]
            svec = jnp.full((16,), s, jnp.int32)

            def chunk(j, carry):
                ocur, started = carry
                og = pl.multiple_of(o + j * 16, 16)
                rv = gid[pl.ds(og, 16)]
                pv = gpos[pl.ds(og, 16)]
                valid = iota < (n - j * 16)
                lanes = rv & (LANES - 1)
                nv = plsc.cumsum(jnp.where(valid, 1, 0))
                rows = lax.rem(ocur + nv - 1, SROWS)
                m = jnp.minimum(n - j * 16, 16)

                def one_entity(_):
                    l0 = jnp.full((16,), lanes[0], jnp.int32)
                    r0 = jnp.full((16,), rows[0], jnp.int32)
                    for k in range(EMBED_DIM // 16):
                        feat = iota + 16 * k
                        v = plsc.load_gather(bufs, [svec, feat, l0])
                        plsc.store_scatter(stag, [r0, feat], v)
                    return 0

                def two_entities(_):
                    hi8 = iota >= 8
                    lv = jnp.where(hi8, lanes[1], lanes[0])
                    rw = jnp.where(hi8, rows[1], rows[0])
                    for k in range(EMBED_DIM // 8):
                        feat = (iota & 7) + 8 * k
                        v = plsc.load_gather(bufs, [svec, feat, lv])
                        plsc.store_scatter(stag, [rw, feat], v)
                    return 0

                def generic(_):
                    for ff in range(EMBED_DIM):
                        fv = jnp.full((16,), ff, jnp.int32)
                        v = plsc.load_gather(bufs, [svec, fv, lanes], mask=valid)
                        plsc.store_scatter(stag, [rows, fv], v, mask=valid)
                    return 0

                lax.switch(jnp.clip(m, 1, 3) - 1, [one_entity, two_entities, generic], 0)
                plsc.store_scatter(stagpos, [rows], pv, mask=valid)
                ocur2 = ocur + nv[15]
                crossed = (ocur2 >> 4) != (ocur >> 4)
                started2 = lax.cond(
                    crossed,
                    lambda st: flush(
                        pl.multiple_of(lax.rem(ocur >> 4, SROWS // 16) * 16, 16),
                        st,
                    ),
                    lambda st: st,
                    started,
                )
                return ocur2, started2

            return lax.fori_loop(0, (n + 15) >> 4, chunk, (ocur, started))

        def main(c, carry):
            f, nseen, ocur, started = carry

            def do(carry):
                f, nseen, ocur, started = carry

                def normal(carry):
                    f, ocur, started = carry
                    s = lax.rem(f, NBUF)

                    def drain(carry):
                        ocur, started = carry
                        cprev = pend[s]
                        pltpu.make_async_copy(
                            table_t_hbm.at[:, pl.ds(0, LANES)],
                            bufs.at[s],
                            fsem.at[s],
                        ).wait()
                        return extract(cprev, s, ocur, started)

                    ocur, started = lax.cond(
                        f >= NBUF, drain, lambda x: x, (ocur, started)
                    )
                    fire(c, s)
                    pend[s] = c
                    return f + 1, ocur, started

                f, ocur, started = lax.cond(
                    nseen >= nfired, normal, lambda x: x, (f, ocur, started)
                )
                return f, nseen + 1, ocur, started

            return lax.cond(
                hist[c] > 0, do, lambda x: x, (f, nseen, ocur, started)
            )

        f, _, ocur, started = lax.fori_loop(
            0,
            tpw,
            main,
            (nfired, jnp.int32(0), jnp.int32(0), jnp.int32(0)),
        )

        def tail(k, carry):
            f0, ocur, started = carry

            def do(carry):
                f0, ocur, started = carry
                s = lax.rem(f0, NBUF)
                cprev = pend[s]
                pltpu.make_async_copy(
                    table_t_hbm.at[:, pl.ds(0, LANES)], bufs.at[s], fsem.at[s]
                ).wait()
                ocur, started = extract(cprev, s, ocur, started)
                return f0 + 1, ocur, started

            return lax.cond(f0 < f, do, lambda x: x, (f0, ocur, started))

        start0 = jnp.maximum(f - NBUF, 0)
        _, ocur, started = lax.fori_loop(0, NBUF, tail, (start0, ocur, started))

        def ragged(st):
            blk = pl.multiple_of(lax.rem(ocur >> 4, SROWS // 16) * 16, 16)
            nvalid = ocur & 15
            pv = stagpos[pl.ds(blk, 16)]
            pv2 = jnp.where(iota < nvalid, pv, BATCH + iota)
            plsc.store_scatter(stagpos, [blk + iota], pv2)
            return flush(blk, st)

        started = lax.cond((ocur & 15) > 0, ragged, lambda st: st, started)

        @pl.when(started >= 1)
        def _():
            wait_one_flush()

    return gather


def kernel(entities, entity_embeddings):
    out = _build_gather()(entities.astype(jnp.int32), entity_embeddings.T)
    return out[:BATCH, :EMBED_DIM]


# speedup vs baseline: 1.0044x; 1.0044x over previous
import functools

import jax
import jax.numpy as jnp
from jax import lax
from jax.experimental import pallas as pl
from jax.experimental.pallas import tpu as pltpu
from jax.experimental.pallas import tpu_sc as plsc

NUM_ENTITIES = 1000000
EMBED_DIM = 64
BATCH = 16384
LANES = 128
NT = (NUM_ENTITIES + LANES - 1) // LANES
NBUF = 6
OUT_PAD = 16
SROWS = 48


@functools.cache
def _build_gather():
    info = plsc.get_sparse_core_info()
    nc, ns = info.num_cores, info.num_subcores
    nw = nc * ns
    tpw = (NT + nw - 1) // nw
    cap = BATCH + tpw * 15 + 16

    mesh = plsc.VectorSubcoreMesh(core_axis_name="c", subcore_axis_name="s")

    @functools.partial(
        pl.kernel,
        mesh=mesh,
        out_type=jax.ShapeDtypeStruct((BATCH + OUT_PAD, LANES), jnp.float32),
        scratch_types=[
            pltpu.VMEM((BATCH,), jnp.int32),
            pltpu.VMEM((BATCH,), jnp.int32),
            pltpu.VMEM((cap,), jnp.int32),
            pltpu.VMEM((cap,), jnp.int32),
            pltpu.VMEM((NBUF, EMBED_DIM, LANES), jnp.float32),
            pltpu.VMEM((SROWS, LANES), jnp.float32),
            pltpu.VMEM((SROWS,), jnp.int32),
            pltpu.SMEM((tpw,), jnp.int32),
            pltpu.SMEM((tpw,), jnp.int32),
            pltpu.SMEM((tpw,), jnp.int32),
            pltpu.SMEM((NBUF,), jnp.int32),
            pltpu.SemaphoreType.DMA((NBUF,)),
            pltpu.SemaphoreType.DMA,
        ],
        compiler_params=pltpu.CompilerParams(needs_layout_passes=False),
    )
    def gather(
        idx_hbm,
        table_t_hbm,
        out_hbm,
        idx_all,
        mpos,
        gid,
        gpos,
        bufs,
        stag,
        stagpos,
        hist,
        goff,
        gcur,
        pend,
        fsem,
        osem,
    ):
        wid = lax.axis_index("s") * nc + lax.axis_index("c")
        lo = wid * tpw
        hi = jnp.minimum(lo + tpw, NT)
        pltpu.sync_copy(idx_hbm, idx_all)

        iota = lax.iota(jnp.int32, 16)
        lane0 = iota < 1

        def p1(g, cnt):
            v = idx_all[pl.ds(pl.multiple_of(g * 16, 8), 16)]
            t = v >> 7
            m = (t >= lo) & (t < hi)
            s = plsc.cumsum(jnp.where(m, 1, 0))
            dst = cnt + s - 1
            plsc.store_scatter(idx_all, [dst], v, mask=m)
            plsc.store_scatter(mpos, [dst], iota + g * 16, mask=m)
            return cnt + s[15]

        cnt = lax.fori_loop(0, BATCH // 16, p1, jnp.int32(0))
        nchunks = (cnt + 15) >> 4

        def p2_clear(c, carry):
            hist[c] = 0
            return carry

        lax.fori_loop(0, tpw, p2_clear, None)

        def p2(g, carry):
            v = idx_all[pl.ds(pl.multiple_of(g * 16, 8), 16)]
            for b in range(16):

                @pl.when(g * 16 + b < cnt)
                def _():
                    t = (v[b] >> 7) - lo
                    hist[t] = hist[t] + 1

            return carry

        lax.fori_loop(0, nchunks, p2, None)

        def fire(c, s):
            t = pl.multiple_of((lo + c) << 7, LANES)
            pltpu.make_async_copy(
                table_t_hbm.at[:, pl.ds(t, LANES)], bufs.at[s], fsem.at[s]
            ).start()

        def p3(c, run):
            goff[c] = run
            gcur[c] = run
            return run + ((hist[c] + 15) & ~15)

        lax.fori_loop(0, tpw, p3, jnp.int32(0))

        def p4(g, carry):
            v = idx_all[pl.ds(pl.multiple_of(g * 16, 8), 16)]
            p = mpos[pl.ds(pl.multiple_of(g * 16, 8), 16)]
            for b in range(16):

                @pl.when(g * 16 + b < cnt)
                def _():
                    t = (v[b] >> 7) - lo
                    slot = gcur[t]
                    gcur[t] = slot + 1
                    sv = jnp.full((16,), slot, jnp.int32)
                    plsc.store_scatter(
                        gid, [sv], jnp.full((16,), v[b], jnp.int32), mask=lane0
                    )
                    plsc.store_scatter(
                        gpos, [sv], jnp.full((16,), p[b], jnp.int32), mask=lane0
                    )

            return carry

        lax.fori_loop(0, nchunks, p4, None)

        def wait_one_flush():
            pltpu.make_async_copy(
                table_t_hbm.at[pl.ds(0, 16), pl.ds(0, LANES)],
                stag.at[pl.ds(0, 16), :],
                osem,
            ).wait()

        def flush(blk, started):
            @pl.when(started >= 1)
            def _():
                wait_one_flush()

            pltpu.make_async_copy(
                stag.at[pl.ds(blk, 16), :],
                out_hbm.at[stagpos.at[pl.ds(blk, 16)]],
                osem,
            ).start()
            return started + 1

        def extract(c, s, ocur, started):
            n = hist[c]
            o = goff[c]
            svec = jnp.full((16,), s, jnp.int32)

            def chunk(j, carry):
                ocur, started = carry
                og = pl.multiple_of(o + j * 16, 16)
                rv = gid[pl.ds(og, 16)]
                pv = gpos[pl.ds(og, 16)]
                valid = iota < (n - j * 16)
                lanes = rv & (LANES - 1)
                nv = plsc.cumsum(jnp.where(valid, 1, 0))
                rows = lax.rem(ocur + nv - 1, SROWS)
                for ff in range(EMBED_DIM):
                    fv = jnp.full((16,), ff, jnp.int32)
                    v = plsc.load_gather(bufs, [svec, fv, lanes], mask=valid)
                    plsc.store_scatter(stag, [rows, fv], v, mask=valid)
                plsc.store_scatter(stagpos, [rows], pv, mask=valid)
                ocur2 = ocur + nv[15]
                crossed = (ocur2 >> 4) != (ocur >> 4)
                started2 = lax.cond(
                    crossed,
                    lambda st: flush(
                        pl.multiple_of(lax.rem(ocur >> 4, SROWS // 16) * 16, 16),
                        st,
                    ),
                    lambda st: st,
                    started,
                )
                return ocur2, started2

            return lax.fori_loop(0, (n + 15) >> 4, chunk, (ocur, started))

        def main(c, carry):
            f, ocur, started = carry

            def do(carry):
                f, ocur, started = carry
                s = lax.rem(f, NBUF)

                def drain(carry):
                    ocur, started = carry
                    cprev = pend[s]
                    pltpu.make_async_copy(
                        table_t_hbm.at[:, pl.ds(0, LANES)], bufs.at[s], fsem.at[s]
                    ).wait()
                    return extract(cprev, s, ocur, started)

                ocur, started = lax.cond(
                    f >= NBUF, drain, lambda x: x, (ocur, started)
                )
                fire(c, s)
                pend[s] = c
                return f + 1, ocur, started

            return lax.cond(hist[c] > 0, do, lambda x: x, (f, ocur, started))

        f, ocur, started = lax.fori_loop(
            0, tpw, main, (jnp.int32(0), jnp.int32(0), jnp.int32(0))
        )

        def tail(k, carry):
            f0, ocur, started = carry

            def do(carry):
                f0, ocur, started = carry
                s = lax.rem(f0, NBUF)
                cprev = pend[s]
                pltpu.make_async_copy(
                    table_t_hbm.at[:, pl.ds(0, LANES)], bufs.at[s], fsem.at[s]
                ).wait()
                ocur, started = extract(cprev, s, ocur, started)
                return f0 + 1, ocur, started

            return lax.cond(f0 < f, do, lambda x: x, (f0, ocur, started))

        start0 = jnp.maximum(f - NBUF, 0)
        _, ocur, started = lax.fori_loop(0, NBUF, tail, (start0, ocur, started))

        def ragged(st):
            blk = pl.multiple_of(lax.rem(ocur >> 4, SROWS // 16) * 16, 16)
            nvalid = ocur & 15
            pv = stagpos[pl.ds(blk, 16)]
            pv2 = jnp.where(iota < nvalid, pv, BATCH + iota)
            plsc.store_scatter(stagpos, [blk + iota], pv2)
            return flush(blk, st)

        started = lax.cond((ocur & 15) > 0, ragged, lambda st: st, started)

        @pl.when(started >= 1)
        def _():
            wait_one_flush()

    return gather


def kernel(entities, entity_embeddings):
    out = _build_gather()(entities.astype(jnp.int32), entity_embeddings.T)
    return out[:BATCH, :EMBED_DIM]
